# Initial kernel scaffold; baseline (speedup 1.0000x reference)
import jax
import jax.numpy as jnp
from jax.experimental import pallas as pl


def kernel(x, edge_index, W1_0, b1_0, W2_0, b2_0, W1_1, b1_1, W2_1, b2_1):
    raise NotImplementedError("write your pallas kernel here")


# speedup vs baseline: 4.6863x; 4.6863x over previous
import functools

import jax
import jax.numpy as jnp
from jax import lax
from jax.experimental import pallas as pl
from jax.experimental.pallas import tpu as pltpu
from jax.experimental.pallas import tpu_sc as plsc

N = 10000
E = 320000
D = 128
NC = 2
NS = 16
NW = NC * NS
EPW = E // NW
CH = 80
ZR = 208
RPS = 624
TAIL = N - NS * RPS


def _make_agg():
    mesh = plsc.VectorSubcoreMesh(core_axis_name="c", subcore_axis_name="s")

    @functools.partial(
        pl.kernel,
        out_type=jax.ShapeDtypeStruct((NC, N, D), jnp.float32),
        mesh=mesh,
        scratch_types=[
            pltpu.VMEM((CH,), jnp.int32),
            pltpu.VMEM((CH,), jnp.int32),
            pltpu.VMEM((CH, D), jnp.float32),
            pltpu.VMEM((ZR, D), jnp.float32),
            pltpu.VMEM_SHARED((N, D), jnp.float32),
            pltpu.SemaphoreType.DMA,
        ],
    )
    def agg(h_hbm, src_hbm, dst_hbm, out_hbm, sidx, didx, rows, zbuf, acc,
            sem):
        cid = lax.axis_index("c")
        sid = lax.axis_index("s")

        def zero_body(i, carry):
            zbuf[i // (D // 16), pl.ds((i % (D // 16)) * 16, 16)] = (
                jnp.zeros((16,), jnp.float32))
            return carry

        lax.fori_loop(0, (ZR * D) // 16, zero_body, 0)
        for j in range(RPS // ZR):
            pltpu.sync_copy(zbuf, acc.at[pl.ds(sid * RPS + j * ZR, ZR)])

        @pl.when(sid == 0)
        def _():
            pltpu.sync_copy(zbuf.at[pl.ds(0, TAIL)],
                            acc.at[pl.ds(NS * RPS, TAIL)])

        plsc.subcore_barrier()

        e0 = (cid * NS + sid) * EPW

        def body(i, carry):
            off = e0 + i * CH
            pltpu.sync_copy(src_hbm.at[pl.ds(off, CH)], sidx)
            pltpu.sync_copy(dst_hbm.at[pl.ds(off, CH)], didx)
            pltpu.async_copy(h_hbm.at[sidx], rows, sem).wait()
            pltpu.sync_copy(rows, acc.at[didx], add=True)
            return carry

        lax.fori_loop(0, EPW // CH, body, 0)
        plsc.subcore_barrier()

        pltpu.sync_copy(acc.at[pl.ds(sid * RPS, RPS)],
                        out_hbm.at[cid, pl.ds(sid * RPS, RPS)])

        @pl.when(sid == 0)
        def _():
            pltpu.sync_copy(acc.at[pl.ds(NS * RPS, TAIL)],
                            out_hbm.at[cid, pl.ds(NS * RPS, TAIL)])

    return agg


_agg = _make_agg()


def _mlp_body(apply_act, h_ref, p0_ref, p1_ref, w1_ref, b1_ref, w2_ref,
              b2_ref, o_ref):
    rst = h_ref[...] + p0_ref[...] + p1_ref[...]
    hh = jnp.dot(rst, w1_ref[...], precision=lax.Precision.HIGHEST,
                 preferred_element_type=jnp.float32) + b1_ref[...]
    hh = jnp.maximum(hh, 0.0)
    out = jnp.dot(hh, w2_ref[...], precision=lax.Precision.HIGHEST,
                  preferred_element_type=jnp.float32) + b2_ref[...]
    if apply_act:
        out = jnp.maximum(out, 0.0)
    o_ref[...] = out


BR = 1000


def _mlp(h, p0, p1, w1, b1, w2, b2, apply_act):
    return pl.pallas_call(
        functools.partial(_mlp_body, apply_act),
        grid=(N // BR,),
        in_specs=[
            pl.BlockSpec((BR, D), lambda i: (i, 0)),
            pl.BlockSpec((BR, D), lambda i: (i, 0)),
            pl.BlockSpec((BR, D), lambda i: (i, 0)),
            pl.BlockSpec((D, D), lambda i: (0, 0)),
            pl.BlockSpec((1, D), lambda i: (0, 0)),
            pl.BlockSpec((D, D), lambda i: (0, 0)),
            pl.BlockSpec((1, D), lambda i: (0, 0)),
        ],
        out_specs=pl.BlockSpec((BR, D), lambda i: (i, 0)),
        out_shape=jax.ShapeDtypeStruct((N, D), jnp.float32),
    )(h, p0, p1, w1, b1.reshape(1, D), w2, b2.reshape(1, D))


def kernel(x, edge_index, W1_0, b1_0, W2_0, b2_0, W1_1, b1_1, W2_1, b2_1):
    ei = edge_index.astype(jnp.int32)
    src, dst = ei[0], ei[1]
    p = _agg(x, src, dst)
    h1 = _mlp(x, p[0], p[1], W1_0, b1_0, W2_0, b2_0, apply_act=True)
    p2 = _agg(h1, src, dst)
    return _mlp(h1, p2[0], p2[1], W1_1, b1_1, W2_1, b2_1, apply_act=False)


# speedup vs baseline: 7.4620x; 1.5923x over previous
import functools

import jax
import jax.numpy as jnp
from jax import lax
from jax.experimental import pallas as pl
from jax.experimental.pallas import tpu as pltpu
from jax.experimental.pallas import tpu_sc as plsc

N = 10000
E = 320000
D = 128
NC = 2
NS = 16
NW = NC * NS
EPW = E // NW
CH = 50
NCHUNK = EPW // CH
NB = 4
IG = 8
NT = NCHUNK // IG
NP = 10240
RPS = NP // NS
ZR = 40


def _make_agg():
    mesh = plsc.VectorSubcoreMesh(core_axis_name="c", subcore_axis_name="s")

    @functools.partial(
        pl.kernel,
        out_type=jax.ShapeDtypeStruct((NC, NP, D), jnp.float32),
        mesh=mesh,
        scratch_types=(
            [
                pltpu.VMEM((2 * IG, CH), jnp.int32),
                pltpu.VMEM((ZR, D), jnp.float32),
                pltpu.VMEM_SHARED((NP, D), jnp.float32),
            ]
            + [pltpu.VMEM((CH, D), jnp.float32) for _ in range(NB)]
            + [pltpu.SemaphoreType.DMA for _ in range(2 * NB)]
        ),
    )
    def agg(h_hbm, eidx_hbm, out_hbm, eidx, zbuf, acc, *rest):
        rows = rest[:NB]
        gsem = rest[NB:2 * NB]
        ssem = rest[2 * NB:]
        cid = lax.axis_index("c")
        sid = lax.axis_index("s")
        wid = cid * NS + sid

        def zero_body(i, carry):
            zbuf[i // (D // 16), pl.ds((i % (D // 16)) * 16, 16)] = (
                jnp.zeros((16,), jnp.float32))
            return carry

        lax.fori_loop(0, (ZR * D) // 16, zero_body, 0)
        for j in range(RPS // ZR):
            pltpu.sync_copy(zbuf, acc.at[pl.ds(sid * RPS + j * ZR, ZR)])

        plsc.subcore_barrier()

        def body(t, carry):
            pltpu.sync_copy(
                eidx_hbm.at[pl.ds((wid * NT + t) * 2 * IG, 2 * IG)], eidx)
            for gg in range(IG // NB):
                c0 = gg * NB
                gd = [
                    pltpu.async_copy(h_hbm.at[eidx.at[c0 + b]], rows[b],
                                     gsem[b])
                    for b in range(NB)
                ]
                sd = []
                for b in range(NB):
                    gd[b].wait()
                    sd.append(pltpu.async_copy(
                        rows[b], acc.at[eidx.at[IG + c0 + b]], ssem[b],
                        add=True))
                for b in range(NB):
                    sd[b].wait()
            return carry

        lax.fori_loop(0, NT, body, 0)
        plsc.subcore_barrier()

        pltpu.sync_copy(acc.at[pl.ds(sid * RPS, RPS)],
                        out_hbm.at[cid, pl.ds(sid * RPS, RPS)])

    return agg


_agg = _make_agg()


def _mlp_body(apply_act, h_ref, p0_ref, p1_ref, w1_ref, b1_ref, w2_ref,
              b2_ref, o_ref):
    rst = h_ref[...] + p0_ref[...] + p1_ref[...]
    hh = jnp.dot(rst, w1_ref[...], precision=lax.Precision.HIGHEST,
                 preferred_element_type=jnp.float32) + b1_ref[...]
    hh = jnp.maximum(hh, 0.0)
    out = jnp.dot(hh, w2_ref[...], precision=lax.Precision.HIGHEST,
                  preferred_element_type=jnp.float32) + b2_ref[...]
    if apply_act:
        out = jnp.maximum(out, 0.0)
    o_ref[...] = out


BR = 1000


def _mlp(h, p0, p1, w1, b1, w2, b2, apply_act):
    return pl.pallas_call(
        functools.partial(_mlp_body, apply_act),
        grid=(N // BR,),
        in_specs=[
            pl.BlockSpec((BR, D), lambda i: (i, 0)),
            pl.BlockSpec((BR, D), lambda i: (i, 0)),
            pl.BlockSpec((BR, D), lambda i: (i, 0)),
            pl.BlockSpec((D, D), lambda i: (0, 0)),
            pl.BlockSpec((1, D), lambda i: (0, 0)),
            pl.BlockSpec((D, D), lambda i: (0, 0)),
            pl.BlockSpec((1, D), lambda i: (0, 0)),
        ],
        out_specs=pl.BlockSpec((BR, D), lambda i: (i, 0)),
        out_shape=jax.ShapeDtypeStruct((N, D), jnp.float32),
    )(h, p0, p1, w1, b1.reshape(1, D), w2, b2.reshape(1, D))


def kernel(x, edge_index, W1_0, b1_0, W2_0, b2_0, W1_1, b1_1, W2_1, b2_1):
    ei = edge_index.astype(jnp.int32)
    e3 = ei.reshape(2, NW, NT, IG, CH)
    eidx = jnp.stack([e3[0], e3[1]], axis=2).reshape(NW * NT * 2 * IG, CH)
    p = _agg(x, eidx)
    h1 = _mlp(x, p[0], p[1], W1_0, b1_0, W2_0, b2_0, apply_act=True)
    p2 = _agg(h1, eidx)
    return _mlp(h1, p2[0], p2[1], W1_1, b1_1, W2_1, b2_1, apply_act=False)


# speedup vs baseline: 8.1134x; 1.0873x over previous
import functools

import jax
import jax.numpy as jnp
from jax import lax
from jax.experimental import pallas as pl
from jax.experimental.pallas import tpu as pltpu
from jax.experimental.pallas import tpu_sc as plsc

N = 10000
E = 320000
D = 128
NC = 2
NS = 16
NW = NC * NS
EPW = E // NW
CH = 25
NCHUNK = EPW // CH
NB = 8
IG = 16
NT = NCHUNK // IG
NP = 10240
RPS = NP // NS
ZR = 40


def _make_agg():
    mesh = plsc.VectorSubcoreMesh(core_axis_name="c", subcore_axis_name="s")

    @functools.partial(
        pl.kernel,
        out_type=jax.ShapeDtypeStruct((NC, NP, D), jnp.float32),
        mesh=mesh,
        scratch_types=(
            [
                pltpu.VMEM((IG, CH), jnp.int32),
                pltpu.VMEM((IG, CH), jnp.int32),
                pltpu.VMEM((ZR, D), jnp.float32),
                pltpu.VMEM_SHARED((NP, D), jnp.float32),
            ]
            + [pltpu.VMEM((CH, D), jnp.float32) for _ in range(NB)]
            + [pltpu.SemaphoreType.DMA for _ in range(2 * NB)]
        ),
    )
    def agg(h_hbm, src_hbm, dst_hbm, out_hbm, sidx, didx, zbuf, acc, *rest):
        rows = rest[:NB]
        gsem = rest[NB:2 * NB]
        ssem = rest[2 * NB:]
        cid = lax.axis_index("c")
        sid = lax.axis_index("s")
        wid = cid * NS + sid

        def zero_body(i, carry):
            zbuf[i // (D // 16), pl.ds((i % (D // 16)) * 16, 16)] = (
                jnp.zeros((16,), jnp.float32))
            return carry

        lax.fori_loop(0, (ZR * D) // 16, zero_body, 0)
        for j in range(RPS // ZR):
            pltpu.sync_copy(zbuf, acc.at[pl.ds(sid * RPS + j * ZR, ZR)])

        plsc.subcore_barrier()

        def body(t, carry):
            r0 = wid * NCHUNK + t * IG
            pltpu.sync_copy(src_hbm.at[pl.ds(r0, IG)], sidx)
            pltpu.sync_copy(dst_hbm.at[pl.ds(r0, IG)], didx)
            for gg in range(IG // NB):
                c0 = gg * NB
                gd = [
                    pltpu.async_copy(h_hbm.at[sidx.at[c0 + b]], rows[b],
                                     gsem[b])
                    for b in range(NB)
                ]
                sd = []
                for b in range(NB):
                    gd[b].wait()
                    sd.append(pltpu.async_copy(
                        rows[b], acc.at[didx.at[c0 + b]], ssem[b], add=True))
                for b in range(NB):
                    sd[b].wait()
            return carry

        lax.fori_loop(0, NT, body, 0)
        plsc.subcore_barrier()

        pltpu.sync_copy(acc.at[pl.ds(sid * RPS, RPS)],
                        out_hbm.at[cid, pl.ds(sid * RPS, RPS)])

    return agg


_agg = _make_agg()


def _mlp_body(apply_act, h_ref, p0_ref, p1_ref, w1_ref, b1_ref, w2_ref,
              b2_ref, o_ref):
    rst = h_ref[...] + p0_ref[0] + p1_ref[0]
    hh = jnp.dot(rst, w1_ref[...],
                 preferred_element_type=jnp.float32) + b1_ref[...]
    hh = jnp.maximum(hh, 0.0)
    out = jnp.dot(hh, w2_ref[...],
                  preferred_element_type=jnp.float32) + b2_ref[...]
    if apply_act:
        out = jnp.maximum(out, 0.0)
    o_ref[...] = out


BR = 1000


def _mlp(h, p, w1, b1, w2, b2, apply_act):
    return pl.pallas_call(
        functools.partial(_mlp_body, apply_act),
        grid=(N // BR,),
        in_specs=[
            pl.BlockSpec((BR, D), lambda i: (i, 0)),
            pl.BlockSpec((1, BR, D), lambda i: (0, i, 0)),
            pl.BlockSpec((1, BR, D), lambda i: (1, i, 0)),
            pl.BlockSpec((D, D), lambda i: (0, 0)),
            pl.BlockSpec((1, D), lambda i: (0, 0)),
            pl.BlockSpec((D, D), lambda i: (0, 0)),
            pl.BlockSpec((1, D), lambda i: (0, 0)),
        ],
        out_specs=pl.BlockSpec((BR, D), lambda i: (i, 0)),
        out_shape=jax.ShapeDtypeStruct((N, D), jnp.float32),
    )(h, p, p, w1, b1.reshape(1, D), w2, b2.reshape(1, D))


def kernel(x, edge_index, W1_0, b1_0, W2_0, b2_0, W1_1, b1_1, W2_1, b2_1):
    ei = edge_index.astype(jnp.int32)
    src = ei[0].reshape(E // CH, CH)
    dst = ei[1].reshape(E // CH, CH)
    p = _agg(x, src, dst)
    h1 = _mlp(x, p, W1_0, b1_0, W2_0, b2_0, apply_act=True)
    p2 = _agg(h1, src, dst)
    return _mlp(h1, p2, W1_1, b1_1, W2_1, b2_1, apply_act=False)
